# baseline (speedup 1.0000x reference)
import jax
import jax.numpy as jnp
from jax import lax
from jax.experimental import pallas as pl
from jax.experimental.pallas import tpu as pltpu
from jax.experimental.pallas import tpu_sc as plsc

_NC = 2
_NS = 16
_NW = _NC * _NS
_LANES = 16
_CHUNK = 64


def kernel(token, token_table, pos_table):
    b, s = token.shape
    v, d = token_table.shape
    n = b * s
    per_w = n // _NW
    n_chunks = per_w // _CHUNK

    def body(tok_hbm, table_hbm, pos_hbm, out_hbm, idx_v, rows_v, pos_v, sem):
        wid = lax.axis_index("s") * _NC + lax.axis_index("c")
        base = wid * per_w
        pos_base = base % s

        def chunk_body(ci, carry):
            off = base + ci * _CHUNK
            poff = pos_base + ci * _CHUNK
            pltpu.sync_copy(tok_hbm.at[pl.ds(off, _CHUNK)], idx_v)
            gather = pltpu.async_copy(table_hbm.at[idx_v], rows_v, sem)
            pltpu.sync_copy(pos_hbm.at[pl.ds(poff, _CHUNK)], pos_v)
            gather.wait()

            def row_body(r, c2):
                for j in range(d // _LANES):
                    sl = pl.ds(j * _LANES, _LANES)
                    rows_v[r, sl] = rows_v[r, sl] + pos_v[r, sl]
                return c2

            lax.fori_loop(0, _CHUNK, row_body, 0)
            pltpu.sync_copy(rows_v, out_hbm.at[pl.ds(off, _CHUNK)])
            return carry

        lax.fori_loop(0, n_chunks, chunk_body, 0)

    mesh = plsc.VectorSubcoreMesh(core_axis_name="c", subcore_axis_name="s")
    run = pl.kernel(
        body,
        mesh=mesh,
        out_type=jax.ShapeDtypeStruct((n, d), jnp.float32),
        scratch_types=[
            pltpu.VMEM((_CHUNK,), jnp.int32),
            pltpu.VMEM((_CHUNK, d), jnp.float32),
            pltpu.VMEM((_CHUNK, d), jnp.float32),
            pltpu.SemaphoreType.DMA,
        ],
    )
    out = run(token.reshape(n).astype(jnp.int32), token_table, pos_table)
    return out.reshape(b, s, d)


# speedup vs baseline: 1.4441x; 1.4441x over previous
import jax
import jax.numpy as jnp
from jax import lax
from jax.experimental import pallas as pl
from jax.experimental.pallas import tpu as pltpu
from jax.experimental.pallas import tpu_sc as plsc

_NC = 2
_NS = 16
_NW = _NC * _NS
_LANES = 16
_CHUNK = 32
_NBUF = 3


def kernel(token, token_table, pos_table):
    b, s = token.shape
    v, d = token_table.shape
    n = b * s
    pos_span = s // _NW
    n_blocks = pos_span // _CHUNK
    chunks = [(h, bi) for h in range(n_blocks) for bi in range(b)]

    def body(tok_hbm, table_hbm, pos_hbm, out_hbm, idx_all, pos_v, *rest):
        rows = rest[:_NBUF]
        gsems = rest[_NBUF:2 * _NBUF]
        ssems = rest[2 * _NBUF:3 * _NBUF]
        wid = lax.axis_index("s") * _NC + lax.axis_index("c")
        pbase = wid * pos_span

        for bi in range(b):
            pltpu.sync_copy(tok_hbm.at[pl.ds(bi * s + pbase, pos_span)],
                            idx_all.at[pl.ds(bi * pos_span, pos_span)])

        def issue_gather(c):
            h, bi = chunks[c]
            buf = c % _NBUF
            idx = idx_all.at[pl.ds(bi * pos_span + h * _CHUNK, _CHUNK)]
            return pltpu.async_copy(table_hbm.at[idx], rows[buf], gsems[buf])

        def issue_store(c):
            h, bi = chunks[c]
            buf = c % _NBUF
            off = bi * s + pbase + h * _CHUNK
            return pltpu.async_copy(rows[buf], out_hbm.at[pl.ds(off, _CHUNK)],
                                    ssems[buf])

        gathers = {}
        stores = {}
        gathers[0] = issue_gather(0)
        for c in range(len(chunks)):
            h, bi = chunks[c]
            buf = c % _NBUF
            if bi == 0:
                pltpu.sync_copy(pos_hbm.at[pl.ds(pbase + h * _CHUNK, _CHUNK)],
                                pos_v)
            nxt = c + 1
            if nxt < len(chunks):
                if nxt >= _NBUF:
                    stores[nxt - _NBUF].wait()
                gathers[nxt] = issue_gather(nxt)
            gathers[c].wait()

            def row_body(r, carry, _buf=buf):
                for j in range(d // _LANES):
                    sl = pl.ds(j * _LANES, _LANES)
                    plsc.addupdate(rows[_buf].at[r, sl], pos_v[r, sl])
                return carry

            lax.fori_loop(0, _CHUNK, row_body, 0)
            stores[c] = issue_store(c)

        for c in range(len(chunks) - _NBUF, len(chunks)):
            if c >= 0:
                stores[c].wait()

    mesh = plsc.VectorSubcoreMesh(core_axis_name="c", subcore_axis_name="s")
    scratch = [
        pltpu.VMEM((b * pos_span,), jnp.int32),
        pltpu.VMEM((_CHUNK, d), jnp.float32),
    ]
    scratch += [pltpu.VMEM((_CHUNK, d), jnp.float32) for _ in range(_NBUF)]
    scratch += [pltpu.SemaphoreType.DMA for _ in range(2 * _NBUF)]
    run = pl.kernel(
        body,
        mesh=mesh,
        out_type=jax.ShapeDtypeStruct((n, d), jnp.float32),
        scratch_types=scratch,
    )
    out = run(token.reshape(n).astype(jnp.int32), token_table, pos_table)
    return out.reshape(b, s, d)
